# baseline (speedup 1.0000x reference)
import jax
import jax.numpy as jnp
import numpy as np
from jax.experimental import pallas as pl
from jax.experimental.pallas import tpu as pltpu

B, T, D, NS, NE, TKS, TKE, NH = 8, 2048, 1024, 8, 16, 2, 2, 4
HD = D // NH
HS = NH * NS
INV = 1.0 / float(np.sqrt(HD))

_CT = (((1,), (1,)), ((), ()))
_CT0 = (((0,), (0,)), ((), ()))
_F32 = jnp.float32


def _dot(a, b, dn=_CT):
    return jax.lax.dot_general(a, b, dn, preferred_element_type=_F32)


def _top2(vals, width):
    ii = jax.lax.broadcasted_iota(jnp.int32, vals.shape, len(vals.shape) - 1)
    m1 = jnp.max(vals, axis=-1, keepdims=True)
    i1 = jnp.min(jnp.where(vals == m1, ii, width), axis=-1, keepdims=True)
    masked = jnp.where(ii == i1, -jnp.inf, vals)
    m2 = jnp.max(masked, axis=-1, keepdims=True)
    i2 = jnp.min(jnp.where(masked == m2, ii, width), axis=-1, keepdims=True)
    return (jnp.concatenate([m1, m2], axis=-1),
            jnp.concatenate([i1, i2], axis=-1))


def _softmax2(v2):
    m = jnp.max(v2, axis=-1, keepdims=True)
    e = jnp.exp(v2 - m)
    return e / jnp.sum(e, axis=-1, keepdims=True)


def _main_kernel(ss_ref, x_ref, Win_ref, bin_ref, lng_ref, lnb_ref,
                 Wq_ref, bq_ref, Wk_ref, bk_ref, Wv_ref, bv_ref,
                 Wr_ref, br_ref, Wo_ref, bo_ref,
                 ridx_ref, ao_ref,
                 qp_s, cb_s, zs_s, vst_s, bvf_s):
    b = pl.program_id(0)

    @pl.when(b == 0)
    def _prep():
        ss = ss_ref[...]
        Win = Win_ref[...]
        Wk = Wk_ref[...]
        bin2 = bin_ref[...]
        bk2 = bk_ref[...]

        rs = jnp.sum(ss * Wr_ref[...][None], axis=-1) + br_ref[0, 0]
        _, ridx = _top2(rs, NS)
        ridx_ref[...] = ridx

        mu = jnp.mean(ss, axis=-1, keepdims=True)
        c = ss - mu
        var = jnp.mean(c * c, axis=-1, keepdims=True)
        sn = c / jnp.sqrt(var + 1e-5) * lng_ref[...][None] + lnb_ref[...][None]
        snf = sn.reshape(B * NS, D)

        q = _dot(snf, Wq_ref[...]) + bq_ref[...]
        kst = _dot(snf, Wk) + bk2
        vst_s[...] = (_dot(snf, Wv_ref[...]) + bv_ref[...]).reshape(B, NS, D)

        kb = _dot(bin2, Wk) + bk2
        bvf_s[...] = _dot(bin2, Wv_ref[...]) + bv_ref[...]

        qps, cbs = [], []
        for h in range(NH):
            sl = slice(h * HD, (h + 1) * HD)
            qh = q[:, sl]
            tt = jnp.dot(qh, Wk[sl, :], preferred_element_type=_F32)
            qps.append(jnp.dot(tt, Win, preferred_element_type=_F32) * INV)
            cbs.append(jnp.sum(qh * kb[:, sl], axis=-1, keepdims=True) * INV)

        for bb in range(B):
            rows = slice(bb * NS, (bb + 1) * NS)
            qp_s[bb] = jnp.concatenate([qps[h][rows] for h in range(NH)],
                                       axis=0)
            cb_s[bb] = jnp.concatenate([cbs[h][rows] for h in range(NH)],
                                       axis=0)
            zb = []
            for h in range(NH):
                sl = slice(h * HD, (h + 1) * HD)
                zb.append(_dot(q[rows, sl], kst[rows, sl]) * INV)
            zs_s[bb] = jnp.concatenate(zb, axis=0)

    xb = x_ref[0]
    qp = qp_s[b]
    zs = zs_s[b]
    S = _dot(qp, xb) + cb_s[b]
    m = jnp.maximum(jnp.max(zs, axis=-1, keepdims=True),
                    jnp.max(S, axis=-1, keepdims=True))
    P = jnp.exp(S - m)
    pst = jnp.exp(zs - m)
    ztok = jnp.sum(P, axis=-1, keepdims=True)
    Z = ztok + jnp.sum(pst, axis=-1, keepdims=True)
    A = jnp.dot(P, xb, preferred_element_type=_F32)
    xbar = A / Z
    wtok = ztok / Z
    pstn = pst / Z

    U = _dot(xbar, Win_ref[...])
    Yv = _dot(U, Wv_ref[...])
    st = jnp.dot(pstn, vst_s[b], preferred_element_type=_F32)
    full = Yv + wtok * bvf_s[...] + st
    r = jax.lax.broadcasted_iota(jnp.int32, (HS, D), 0)
    d = jax.lax.broadcasted_iota(jnp.int32, (HS, D), 1)
    hmask = (r // NS) == (d // HD)
    ao_pre = jnp.sum(jnp.where(hmask, full, 0.0).reshape(NH, NS, D), axis=0)
    ao_ref[0] = _dot(ao_pre, Wo_ref[...]) + bo_ref[...]


def _route_out_kernel(ao_ref, ss_ref, We_ref, be_ref, Wg_ref, bg_ref,
                      Wsg_ref, bsg_ref, Wsp_ref, bsp_ref, Wout_ref, bout_ref,
                      eidx_ref, gw_ref, sidx_ref, sw_ref, ns_ref, out_ref,
                      nsT_s):
    c = pl.program_id(0)

    @pl.when(c == 0)
    def _route():
        ao = ao_ref[...]
        aof = ao.reshape(B * NS, D)
        ssf = ss_ref[...].reshape(B * NS, D)
        am = jnp.mean(ao, axis=1)
        gl = _dot(am, Wg_ref[...]) + bg_ref[...]
        gval, eidx = _top2(gl, NE)
        gw = _softmax2(gval)
        eidx_ref[...] = eidx
        gw_ref[...] = gw
        gs = jnp.sum(gw, axis=-1, keepdims=True)

        r_i = jax.lax.broadcasted_iota(jnp.int32, (B * NS, B), 0)
        b_i = jax.lax.broadcasted_iota(jnp.int32, (B * NS, B), 1)
        rb = (r_i // NS == b_i).astype(_F32)
        smod = (jax.lax.broadcasted_iota(jnp.int32, (B * NS, 1), 0) % NS
                ).astype(_F32)

        gs_rows = jnp.dot(rb, gs, preferred_element_type=_F32)
        eof = (_dot(aof, We_ref[...]) + be_ref[...]) * gs_rows
        sscore = jnp.sum(eof.reshape(B, NS, D) * Wsg_ref[...][None],
                         axis=-1) + bsg_ref[0, 0]
        sval, sidx = _top2(sscore, NS)
        sw = _softmax2(sval)
        sidx_ref[...] = sidx
        sw_ref[...] = sw

        sidxf = sidx.astype(_F32)
        ns = ssf
        for i in range(TKS):
            target = jnp.dot(rb, sidxf[:, i:i + 1],
                             preferred_element_type=_F32)
            rmask = (smod == target).astype(_F32)
            sel = _dot(rb, eof * rmask, _CT0)
            upd = jnp.tanh(_dot(sel, Wsp_ref[...]) + bsp_ref[...])
            ssel = _dot(rb, ssf * rmask, _CT0)
            newv = 0.7 * ssel + 0.3 * sw[:, i:i + 1] * upd
            ns = ns * (1.0 - rmask) + jnp.dot(
                rb, newv, preferred_element_type=_F32) * rmask
        ns3 = ns.reshape(B, NS, D)
        ns_ref[...] = ns3
        for s in range(NS):
            nsT_s[s] = ns3[:, s, :]

    part = _dot(nsT_s[c], Wout_ref[...])

    @pl.when(c == 0)
    def _():
        out_ref[...] = part + bout_ref[...]

    @pl.when(c != 0)
    def _():
        out_ref[...] += part


def kernel(x, state, W_in, b_in, ln_g, ln_b, Wq, bq, Wk, bk, Wv, bv, Wo, bo,
           We, be, Wg, bg, Wr, br, Wsg, bsg, Wsp, bsp, Wout, bout):
    f32 = jnp.float32
    ss = state.reshape(B, NS, D)
    b2 = lambda v: v.reshape(1, -1)
    cst = lambda shape: pl.BlockSpec(shape, lambda i: tuple(0 for _ in shape))

    ridx, ao = pl.pallas_call(
        _main_kernel,
        grid=(B,),
        in_specs=[
            cst((B, NS, D)),
            pl.BlockSpec((1, T, D), lambda b: (b, 0, 0)),
            cst((D, D)), cst((1, D)), cst((1, D)), cst((1, D)),
            cst((D, D)), cst((1, D)), cst((D, D)), cst((1, D)),
            cst((D, D)), cst((1, D)), cst((1, D)), cst((1, 1)),
            cst((D, D)), cst((1, D)),
        ],
        out_specs=(
            cst((B, TKS)),
            pl.BlockSpec((1, NS, D), lambda b: (b, 0, 0)),
        ),
        out_shape=(
            jax.ShapeDtypeStruct((B, TKS), jnp.int32),
            jax.ShapeDtypeStruct((B, NS, D), f32),
        ),
        scratch_shapes=[
            pltpu.VMEM((B, HS, D), f32),
            pltpu.VMEM((B, HS, 1), f32),
            pltpu.VMEM((B, HS, NS), f32),
            pltpu.VMEM((B, NS, D), f32),
            pltpu.VMEM((1, D), f32),
        ],
    )(ss, x, W_in, b2(b_in), b2(ln_g), b2(ln_b), Wq, b2(bq), Wk, b2(bk),
      Wv, b2(bv), Wr, b2(br), Wo, b2(bo))

    eidx, gw, sidx, sw, ns, out = pl.pallas_call(
        _route_out_kernel,
        grid=(NS,),
        in_specs=[
            cst((B, NS, D)), cst((B, NS, D)),
            cst((D, D)), cst((1, D)),
            cst((NE, D)), cst((1, NE)),
            cst((1, D)), cst((1, 1)),
            cst((D, D)), cst((1, D)),
            pl.BlockSpec((Wout.shape[0], D), lambda c: (0, c)),
            cst((1, Wout.shape[0])),
        ],
        out_specs=(
            cst((B, TKE)), cst((B, TKE)), cst((B, TKS)), cst((B, TKS)),
            cst((B, NS, D)), cst((B, Wout.shape[0])),
        ),
        out_shape=(
            jax.ShapeDtypeStruct((B, TKE), jnp.int32),
            jax.ShapeDtypeStruct((B, TKE), f32),
            jax.ShapeDtypeStruct((B, TKS), jnp.int32),
            jax.ShapeDtypeStruct((B, TKS), f32),
            jax.ShapeDtypeStruct((B, NS, D), f32),
            jax.ShapeDtypeStruct((B, Wout.shape[0]), f32),
        ),
        scratch_shapes=[pltpu.VMEM((NS, B, D), f32)],
    )(ao, ss, We, b2(be), Wg, b2(bg), Wsg, b2(bsg), Wsp, b2(bsp),
      Wout, b2(bout))

    nsf = ns.reshape(B, NS * D)
    return out, ridx, eidx, sidx, sw, gw, nsf


# speedup vs baseline: 1.0807x; 1.0807x over previous
import jax
import jax.numpy as jnp
import numpy as np
from jax.experimental import pallas as pl
from jax.experimental.pallas import tpu as pltpu

B, T, D, NS, NE, TKS, TKE, NH = 8, 2048, 1024, 8, 16, 2, 2, 4
HD = D // NH
HS = NH * NS
WCH = NS * D // 4
INV = 1.0 / float(np.sqrt(HD))

_CT = (((1,), (1,)), ((), ()))
_CT0 = (((0,), (0,)), ((), ()))
_F32 = jnp.float32


def _dot(a, b, dn=_CT):
    return jax.lax.dot_general(a, b, dn, preferred_element_type=_F32)


def _top2(vals, width):
    ii = jax.lax.broadcasted_iota(jnp.int32, vals.shape, len(vals.shape) - 1)
    m1 = jnp.max(vals, axis=-1, keepdims=True)
    i1 = jnp.min(jnp.where(vals == m1, ii, width), axis=-1, keepdims=True)
    masked = jnp.where(ii == i1, -jnp.inf, vals)
    m2 = jnp.max(masked, axis=-1, keepdims=True)
    i2 = jnp.min(jnp.where(masked == m2, ii, width), axis=-1, keepdims=True)
    return (jnp.concatenate([m1, m2], axis=-1),
            jnp.concatenate([i1, i2], axis=-1))


def _softmax2(v2):
    m = jnp.max(v2, axis=-1, keepdims=True)
    e = jnp.exp(v2 - m)
    return e / jnp.sum(e, axis=-1, keepdims=True)


def _main_kernel(ss_ref, x_ref, Win_ref, bin_ref, lng_ref, lnb_ref,
                 Wq_ref, bq_ref, Wk_ref, bk_ref, Wv_ref, bv_ref,
                 Wr_ref, br_ref,
                 ridx_ref, xbar_ref, wtok_ref, pst_ref, vst_ref, bvf_ref,
                 qp_s, cb_s, zs_s):
    b = pl.program_id(0)

    @pl.when(b == 0)
    def _prep():
        ss = ss_ref[...]
        Win = Win_ref[...]
        Wk = Wk_ref[...]
        bin2 = bin_ref[...]
        bk2 = bk_ref[...]

        rs = jnp.sum(ss * Wr_ref[...][None], axis=-1) + br_ref[0, 0]
        _, ridx = _top2(rs, NS)
        ridx_ref[...] = ridx

        mu = jnp.mean(ss, axis=-1, keepdims=True)
        c = ss - mu
        var = jnp.mean(c * c, axis=-1, keepdims=True)
        sn = c / jnp.sqrt(var + 1e-5) * lng_ref[...][None] + lnb_ref[...][None]
        snf = sn.reshape(B * NS, D)

        q = _dot(snf, Wq_ref[...]) + bq_ref[...]
        kst = _dot(snf, Wk) + bk2
        vst_ref[...] = (_dot(snf, Wv_ref[...]) + bv_ref[...]).reshape(B, NS, D)

        kb = _dot(bin2, Wk) + bk2
        bvf_ref[...] = _dot(bin2, Wv_ref[...]) + bv_ref[...]

        qps, cbs = [], []
        for h in range(NH):
            sl = slice(h * HD, (h + 1) * HD)
            qh = q[:, sl]
            tt = jnp.dot(qh, Wk[sl, :], preferred_element_type=_F32)
            qps.append(jnp.dot(tt, Win, preferred_element_type=_F32) * INV)
            cbs.append(jnp.sum(qh * kb[:, sl], axis=-1, keepdims=True) * INV)

        for bb in range(B):
            rows = slice(bb * NS, (bb + 1) * NS)
            qp_s[bb] = jnp.concatenate([qps[h][rows] for h in range(NH)],
                                       axis=0)
            cb_s[bb] = jnp.concatenate([cbs[h][rows] for h in range(NH)],
                                       axis=0)
            zb = []
            for h in range(NH):
                sl = slice(h * HD, (h + 1) * HD)
                zb.append(_dot(q[rows, sl], kst[rows, sl]) * INV)
            zs_s[bb] = jnp.concatenate(zb, axis=0)

    xb = x_ref[0]
    qp = qp_s[b]
    zs = zs_s[b]
    S = _dot(qp, xb) + cb_s[b]
    m = jnp.maximum(jnp.max(zs, axis=-1, keepdims=True),
                    jnp.max(S, axis=-1, keepdims=True))
    P = jnp.exp(S - m)
    pst = jnp.exp(zs - m)
    ztok = jnp.sum(P, axis=-1, keepdims=True)
    Z = ztok + jnp.sum(pst, axis=-1, keepdims=True)
    A = jnp.dot(P, xb, preferred_element_type=_F32)
    xbar_ref[0] = A / Z
    wtok_ref[0] = ztok / Z
    pst_ref[0] = pst / Z


def _route_out_kernel(xbar_ref, wtok_ref, pst_ref, vst_ref, bvf_ref,
                      Win_ref, Wv_ref, Wo_ref, bo_ref,
                      ss_ref, We_ref, be_ref, Wg_ref, bg_ref,
                      Wsg_ref, bsg_ref, Wsp_ref, bsp_ref, Wout_ref, bout_ref,
                      eidx_ref, gw_ref, sidx_ref, sw_ref, ns_ref, out_ref,
                      nsT_s):
    c = pl.program_id(0)

    @pl.when(c == 0)
    def _route():
        xf = xbar_ref[...].reshape(B * HS, D)
        U = _dot(xf, Win_ref[...])
        Yv = _dot(U, Wv_ref[...])
        wt = wtok_ref[...].reshape(B * HS, 1)
        st = jnp.concatenate(
            [jnp.dot(pst_ref[bb], vst_ref[bb], preferred_element_type=_F32)
             for bb in range(B)], axis=0)
        full = Yv + wt * bvf_ref[...] + st
        r0 = jax.lax.broadcasted_iota(jnp.int32, (B * HS, D), 0)
        d0 = jax.lax.broadcasted_iota(jnp.int32, (B * HS, D), 1)
        hm = ((r0 % HS) // NS) == (d0 // HD)
        ao_pre = jnp.sum(jnp.where(hm, full, 0.0).reshape(B, NH, NS, D),
                         axis=1)
        aof = (_dot(ao_pre.reshape(B * NS, D), Wo_ref[...])
               + bo_ref[...])
        ao = aof.reshape(B, NS, D)

        ssf = ss_ref[...].reshape(B * NS, D)
        am = jnp.mean(ao, axis=1)
        gl = _dot(am, Wg_ref[...]) + bg_ref[...]
        gval, eidx = _top2(gl, NE)
        gw = _softmax2(gval)
        eidx_ref[...] = eidx
        gw_ref[...] = gw
        gs = jnp.sum(gw, axis=-1, keepdims=True)

        r_i = jax.lax.broadcasted_iota(jnp.int32, (B * NS, B), 0)
        b_i = jax.lax.broadcasted_iota(jnp.int32, (B * NS, B), 1)
        rb = (r_i // NS == b_i).astype(_F32)
        smod = (jax.lax.broadcasted_iota(jnp.int32, (B * NS, 1), 0) % NS
                ).astype(_F32)

        gs_rows = jnp.dot(rb, gs, preferred_element_type=_F32)
        eof = (_dot(aof, We_ref[...]) + be_ref[...]) * gs_rows
        sscore = jnp.sum(eof.reshape(B, NS, D) * Wsg_ref[...][None],
                         axis=-1) + bsg_ref[0, 0]
        sval, sidx = _top2(sscore, NS)
        sw = _softmax2(sval)
        sidx_ref[...] = sidx
        sw_ref[...] = sw

        sidxf = sidx.astype(_F32)
        ns = ssf
        for i in range(TKS):
            target = jnp.dot(rb, sidxf[:, i:i + 1],
                             preferred_element_type=_F32)
            rmask = (smod == target).astype(_F32)
            sel = _dot(rb, eof * rmask, _CT0)
            upd = jnp.tanh(_dot(sel, Wsp_ref[...]) + bsp_ref[...])
            ssel = _dot(rb, ssf * rmask, _CT0)
            newv = 0.7 * ssel + 0.3 * sw[:, i:i + 1] * upd
            ns = ns * (1.0 - rmask) + jnp.dot(
                rb, newv, preferred_element_type=_F32) * rmask
        ns3 = ns.reshape(B, NS, D)
        ns_ref[...] = ns3
        nslots = WCH // D
        for cc in range(NS * D // WCH):
            nsT_s[cc] = jnp.concatenate(
                [ns3[:, nslots * cc + j, :] for j in range(nslots)], axis=1)

    part = _dot(nsT_s[c], Wout_ref[...])

    @pl.when(c == 0)
    def _():
        out_ref[...] = part + bout_ref[...]

    @pl.when(c != 0)
    def _():
        out_ref[...] += part


def kernel(x, state, W_in, b_in, ln_g, ln_b, Wq, bq, Wk, bk, Wv, bv, Wo, bo,
           We, be, Wg, bg, Wr, br, Wsg, bsg, Wsp, bsp, Wout, bout):
    f32 = jnp.float32
    ss = state.reshape(B, NS, D)
    b2 = lambda v: v.reshape(1, -1)
    cst = lambda shape: pl.BlockSpec(shape, lambda i: tuple(0 for _ in shape))

    ridx, xbar, wtok, pst, vst, bvf = pl.pallas_call(
        _main_kernel,
        grid=(B,),
        in_specs=[
            cst((B, NS, D)),
            pl.BlockSpec((1, T, D), lambda b: (b, 0, 0)),
            cst((D, D)), cst((1, D)), cst((1, D)), cst((1, D)),
            cst((D, D)), cst((1, D)), cst((D, D)), cst((1, D)),
            cst((D, D)), cst((1, D)), cst((1, D)), cst((1, 1)),
        ],
        out_specs=(
            cst((B, TKS)),
            pl.BlockSpec((1, HS, D), lambda b: (b, 0, 0)),
            pl.BlockSpec((1, HS, 1), lambda b: (b, 0, 0)),
            pl.BlockSpec((1, HS, NS), lambda b: (b, 0, 0)),
            cst((B, NS, D)),
            cst((1, D)),
        ),
        out_shape=(
            jax.ShapeDtypeStruct((B, TKS), jnp.int32),
            jax.ShapeDtypeStruct((B, HS, D), f32),
            jax.ShapeDtypeStruct((B, HS, 1), f32),
            jax.ShapeDtypeStruct((B, HS, NS), f32),
            jax.ShapeDtypeStruct((B, NS, D), f32),
            jax.ShapeDtypeStruct((1, D), f32),
        ),
        scratch_shapes=[
            pltpu.VMEM((B, HS, D), f32),
            pltpu.VMEM((B, HS, 1), f32),
            pltpu.VMEM((B, HS, NS), f32),
        ],
    )(ss, x, W_in, b2(b_in), b2(ln_g), b2(ln_b), Wq, b2(bq), Wk, b2(bk),
      Wv, b2(bv), Wr, b2(br))

    eidx, gw, sidx, sw, ns, out = pl.pallas_call(
        _route_out_kernel,
        grid=(NS * D // WCH,),
        in_specs=[
            cst((B, HS, D)), cst((B, HS, 1)), cst((B, HS, NS)),
            cst((B, NS, D)), cst((1, D)),
            cst((D, D)), cst((D, D)), cst((D, D)), cst((1, D)),
            cst((B, NS, D)),
            cst((D, D)), cst((1, D)),
            cst((NE, D)), cst((1, NE)),
            cst((1, D)), cst((1, 1)),
            cst((D, D)), cst((1, D)),
            pl.BlockSpec((Wout.shape[0], WCH), lambda c: (0, c)),
            cst((1, Wout.shape[0])),
        ],
        out_specs=(
            cst((B, TKE)), cst((B, TKE)), cst((B, TKS)), cst((B, TKS)),
            cst((B, NS, D)), cst((B, Wout.shape[0])),
        ),
        out_shape=(
            jax.ShapeDtypeStruct((B, TKE), jnp.int32),
            jax.ShapeDtypeStruct((B, TKE), f32),
            jax.ShapeDtypeStruct((B, TKS), jnp.int32),
            jax.ShapeDtypeStruct((B, TKS), f32),
            jax.ShapeDtypeStruct((B, NS, D), f32),
            jax.ShapeDtypeStruct((B, Wout.shape[0]), f32),
        ),
        scratch_shapes=[pltpu.VMEM((NS * D // WCH, B, WCH), f32)],
    )(xbar, wtok, pst, vst, bvf, W_in, Wv, Wo, b2(bo), ss,
      We, b2(be), Wg, b2(bg), Wsg, b2(bsg), Wsp, b2(bsp),
      Wout, b2(bout))

    nsf = ns.reshape(B, NS * D)
    return out, ridx, eidx, sidx, sw, gw, nsf


# speedup vs baseline: 1.1022x; 1.0199x over previous
import jax
import jax.numpy as jnp
import numpy as np
from jax.experimental import pallas as pl
from jax.experimental.pallas import tpu as pltpu

B, T, D, NS, NE, TKS, TKE, NH = 8, 2048, 1024, 8, 16, 2, 2, 4
HD = D // NH
HS = NH * NS
WCH = NS * D // 4
GB = 2
INV = 1.0 / float(np.sqrt(HD))

_CT = (((1,), (1,)), ((), ()))
_CT0 = (((0,), (0,)), ((), ()))
_F32 = jnp.float32


def _dot(a, b, dn=_CT):
    return jax.lax.dot_general(a, b, dn, preferred_element_type=_F32)


def _top2(vals, width):
    ii = jax.lax.broadcasted_iota(jnp.int32, vals.shape, len(vals.shape) - 1)
    m1 = jnp.max(vals, axis=-1, keepdims=True)
    i1 = jnp.min(jnp.where(vals == m1, ii, width), axis=-1, keepdims=True)
    masked = jnp.where(ii == i1, -jnp.inf, vals)
    m2 = jnp.max(masked, axis=-1, keepdims=True)
    i2 = jnp.min(jnp.where(masked == m2, ii, width), axis=-1, keepdims=True)
    return (jnp.concatenate([m1, m2], axis=-1),
            jnp.concatenate([i1, i2], axis=-1))


def _softmax2(v2):
    m = jnp.max(v2, axis=-1, keepdims=True)
    e = jnp.exp(v2 - m)
    return e / jnp.sum(e, axis=-1, keepdims=True)


def _main_kernel(ss_ref, x_ref, Win_ref, bin_ref, lng_ref, lnb_ref,
                 Wq_ref, bq_ref, Wk_ref, bk_ref,
                 Wr_ref, br_ref,
                 ridx_ref, xbar_ref, wtok_ref, pst_ref, sn_ref,
                 qp_s, cb_s, zs_s):
    b = pl.program_id(0)

    @pl.when(b == 0)
    def _prep():
        ss = ss_ref[...]
        Win = Win_ref[...]
        Wk = Wk_ref[...]
        bin2 = bin_ref[...]
        bk2 = bk_ref[...]

        rs = jnp.sum(ss * Wr_ref[...][None], axis=-1) + br_ref[0, 0]
        _, ridx = _top2(rs, NS)
        ridx_ref[...] = ridx

        mu = jnp.mean(ss, axis=-1, keepdims=True)
        c = ss - mu
        var = jnp.mean(c * c, axis=-1, keepdims=True)
        sn = c / jnp.sqrt(var + 1e-5) * lng_ref[...][None] + lnb_ref[...][None]
        snf = sn.reshape(B * NS, D)

        sn_ref[...] = sn
        q = _dot(snf, Wq_ref[...]) + bq_ref[...]
        kst = _dot(snf, Wk) + bk2

        kb = _dot(bin2, Wk) + bk2

        qps, cbs = [], []
        for h in range(NH):
            sl = slice(h * HD, (h + 1) * HD)
            qh = q[:, sl]
            tt = jnp.dot(qh, Wk[sl, :], preferred_element_type=_F32)
            qps.append(jnp.dot(tt, Win, preferred_element_type=_F32) * INV)
            cbs.append(jnp.sum(qh * kb[:, sl], axis=-1, keepdims=True) * INV)

        for bb in range(B):
            rows = slice(bb * NS, (bb + 1) * NS)
            qp_s[bb] = jnp.concatenate([qps[h][rows] for h in range(NH)],
                                       axis=0)
            cb_s[bb] = jnp.concatenate([cbs[h][rows] for h in range(NH)],
                                       axis=0)
            zb = []
            for h in range(NH):
                sl = slice(h * HD, (h + 1) * HD)
                zb.append(_dot(q[rows, sl], kst[rows, sl]) * INV)
            zs_s[bb] = jnp.concatenate(zb, axis=0)

    for g in range(GB):
        bb = b * GB + g
        xb = x_ref[g]
        qp = qp_s[bb]
        zs = zs_s[bb]
        S = _dot(qp, xb) + cb_s[bb]
        m = jnp.maximum(jnp.max(zs, axis=-1, keepdims=True),
                        jnp.max(S, axis=-1, keepdims=True))
        P = jnp.exp(S - m)
        pst = jnp.exp(zs - m)
        ztok = jnp.sum(P, axis=-1, keepdims=True)
        Z = ztok + jnp.sum(pst, axis=-1, keepdims=True)
        A = jnp.dot(P, xb, preferred_element_type=_F32)
        xbar_ref[g] = A / Z
        wtok_ref[g] = ztok / Z
        pst_ref[g] = pst / Z


def _route_out_kernel(xbar_ref, wtok_ref, pst_ref, sn_ref, bin_ref, bv_ref,
                      Win_ref, Wv_ref, Wo_ref, bo_ref,
                      ss_ref, We_ref, be_ref, Wg_ref, bg_ref,
                      Wsg_ref, bsg_ref, Wsp_ref, bsp_ref, Wout_ref, bout_ref,
                      eidx_ref, gw_ref, sidx_ref, sw_ref, ns_ref, out_ref,
                      nsT_s):
    c = pl.program_id(0)

    @pl.when(c == 0)
    def _route():
        vst = (_dot(sn_ref[...].reshape(B * NS, D), Wv_ref[...])
               + bv_ref[...]).reshape(B, NS, D)
        bvf = _dot(bin_ref[...], Wv_ref[...]) + bv_ref[...]
        xf = xbar_ref[...].reshape(B * HS, D)
        U = _dot(xf, Win_ref[...])
        Yv = _dot(U, Wv_ref[...])
        wt = wtok_ref[...].reshape(B * HS, 1)
        st = jnp.concatenate(
            [jnp.dot(pst_ref[bb], vst[bb], preferred_element_type=_F32)
             for bb in range(B)], axis=0)
        full = Yv + wt * bvf + st
        r0 = jax.lax.broadcasted_iota(jnp.int32, (B * HS, D), 0)
        d0 = jax.lax.broadcasted_iota(jnp.int32, (B * HS, D), 1)
        hm = ((r0 % HS) // NS) == (d0 // HD)
        ao_pre = jnp.sum(jnp.where(hm, full, 0.0).reshape(B, NH, NS, D),
                         axis=1)
        aof = (_dot(ao_pre.reshape(B * NS, D), Wo_ref[...])
               + bo_ref[...])
        ao = aof.reshape(B, NS, D)

        ssf = ss_ref[...].reshape(B * NS, D)
        am = jnp.mean(ao, axis=1)
        gl = _dot(am, Wg_ref[...]) + bg_ref[...]
        gval, eidx = _top2(gl, NE)
        gw = _softmax2(gval)
        eidx_ref[...] = eidx
        gw_ref[...] = gw
        gs = jnp.sum(gw, axis=-1, keepdims=True)

        r_i = jax.lax.broadcasted_iota(jnp.int32, (B * NS, B), 0)
        b_i = jax.lax.broadcasted_iota(jnp.int32, (B * NS, B), 1)
        rb = (r_i // NS == b_i).astype(_F32)
        smod = (jax.lax.broadcasted_iota(jnp.int32, (B * NS, 1), 0) % NS
                ).astype(_F32)

        gs_rows = jnp.dot(rb, gs, preferred_element_type=_F32)
        eof = (_dot(aof, We_ref[...]) + be_ref[...]) * gs_rows
        sscore = jnp.sum(eof.reshape(B, NS, D) * Wsg_ref[...][None],
                         axis=-1) + bsg_ref[0, 0]
        sval, sidx = _top2(sscore, NS)
        sw = _softmax2(sval)
        sidx_ref[...] = sidx
        sw_ref[...] = sw

        sidxf = sidx.astype(_F32)
        ns = ssf
        for i in range(TKS):
            target = jnp.dot(rb, sidxf[:, i:i + 1],
                             preferred_element_type=_F32)
            rmask = (smod == target).astype(_F32)
            sel = _dot(rb, eof * rmask, _CT0)
            upd = jnp.tanh(_dot(sel, Wsp_ref[...]) + bsp_ref[...])
            ssel = _dot(rb, ssf * rmask, _CT0)
            newv = 0.7 * ssel + 0.3 * sw[:, i:i + 1] * upd
            ns = ns * (1.0 - rmask) + jnp.dot(
                rb, newv, preferred_element_type=_F32) * rmask
        ns3 = ns.reshape(B, NS, D)
        ns_ref[...] = ns3
        nslots = WCH // D
        for cc in range(NS * D // WCH):
            nsT_s[cc] = jnp.concatenate(
                [ns3[:, nslots * cc + j, :] for j in range(nslots)], axis=1)

    part = _dot(nsT_s[c], Wout_ref[...])

    @pl.when(c == 0)
    def _():
        out_ref[...] = part + bout_ref[...]

    @pl.when(c != 0)
    def _():
        out_ref[...] += part


def kernel(x, state, W_in, b_in, ln_g, ln_b, Wq, bq, Wk, bk, Wv, bv, Wo, bo,
           We, be, Wg, bg, Wr, br, Wsg, bsg, Wsp, bsp, Wout, bout):
    f32 = jnp.float32
    ss = state.reshape(B, NS, D)
    b2 = lambda v: v.reshape(1, -1)
    cst = lambda shape: pl.BlockSpec(shape, lambda i: tuple(0 for _ in shape))

    ridx, xbar, wtok, pst, sn = pl.pallas_call(
        _main_kernel,
        grid=(B // GB,),
        in_specs=[
            cst((B, NS, D)),
            pl.BlockSpec((GB, T, D), lambda b: (b, 0, 0)),
            cst((D, D)), cst((1, D)), cst((1, D)), cst((1, D)),
            cst((D, D)), cst((1, D)), cst((D, D)), cst((1, D)),
            cst((1, D)), cst((1, 1)),
        ],
        out_specs=(
            cst((B, TKS)),
            pl.BlockSpec((GB, HS, D), lambda b: (b, 0, 0)),
            pl.BlockSpec((GB, HS, 1), lambda b: (b, 0, 0)),
            pl.BlockSpec((GB, HS, NS), lambda b: (b, 0, 0)),
            cst((B, NS, D)),
        ),
        out_shape=(
            jax.ShapeDtypeStruct((B, TKS), jnp.int32),
            jax.ShapeDtypeStruct((B, HS, D), f32),
            jax.ShapeDtypeStruct((B, HS, 1), f32),
            jax.ShapeDtypeStruct((B, HS, NS), f32),
            jax.ShapeDtypeStruct((B, NS, D), f32),
        ),
        scratch_shapes=[
            pltpu.VMEM((B, HS, D), f32),
            pltpu.VMEM((B, HS, 1), f32),
            pltpu.VMEM((B, HS, NS), f32),
        ],
    )(ss, x, W_in, b2(b_in), b2(ln_g), b2(ln_b), Wq, b2(bq), Wk, b2(bk),
      Wr, b2(br))

    eidx, gw, sidx, sw, ns, out = pl.pallas_call(
        _route_out_kernel,
        grid=(NS * D // WCH,),
        in_specs=[
            cst((B, HS, D)), cst((B, HS, 1)), cst((B, HS, NS)),
            cst((B, NS, D)), cst((1, D)), cst((1, D)),
            cst((D, D)), cst((D, D)), cst((D, D)), cst((1, D)),
            cst((B, NS, D)),
            cst((D, D)), cst((1, D)),
            cst((NE, D)), cst((1, NE)),
            cst((1, D)), cst((1, 1)),
            cst((D, D)), cst((1, D)),
            pl.BlockSpec((Wout.shape[0], WCH), lambda c: (0, c)),
            cst((1, Wout.shape[0])),
        ],
        out_specs=(
            cst((B, TKE)), cst((B, TKE)), cst((B, TKS)), cst((B, TKS)),
            cst((B, NS, D)), cst((B, Wout.shape[0])),
        ),
        out_shape=(
            jax.ShapeDtypeStruct((B, TKE), jnp.int32),
            jax.ShapeDtypeStruct((B, TKE), f32),
            jax.ShapeDtypeStruct((B, TKS), jnp.int32),
            jax.ShapeDtypeStruct((B, TKS), f32),
            jax.ShapeDtypeStruct((B, NS, D), f32),
            jax.ShapeDtypeStruct((B, Wout.shape[0]), f32),
        ),
        scratch_shapes=[pltpu.VMEM((NS * D // WCH, B, WCH), f32)],
    )(xbar, wtok, pst, sn, b2(b_in), b2(bv), W_in, Wv, Wo, b2(bo), ss,
      We, b2(be), Wg, b2(bg), Wsg, b2(bsg), Wsp, b2(bsp),
      Wout, b2(bout))

    nsf = ns.reshape(B, NS * D)
    return out, ridx, eidx, sidx, sw, gw, nsf
